# baseline (speedup 1.0000x reference)
import functools

import jax
import jax.numpy as jnp
from jax import lax
from jax.experimental import pallas as pl
from jax.experimental.pallas import tpu as pltpu
from jax.experimental.pallas import tpu_sc as plsc

N = 10000
N_PAD = 10240
E = 320000
NL = 200000
IN_CH = 128
F2 = 192
OUT_CH = 64

NW = 32
EPW = E // NW
CH = 80
K = EPW // CH
EPT = E // 16
K2 = EPT // CH
RPS = N_PAD // 16
F2H = F2 // 2

CD = 64
WPD = 6272
KD = WPD // CD
NL_PAD = NW * WPD

BR = 512
TGRID = N_PAD // BR

_MESH = plsc.VectorSubcoreMesh(core_axis_name="c", subcore_axis_name="s")

GRID_PTS = [-2.0 + j * (4.0 / 7.0) for j in range(8)]
INV_DENOM = 7.0 / 4.0


@functools.partial(
    pl.kernel,
    out_type=jax.ShapeDtypeStruct((2 * N_PAD,), jnp.float32),
    mesh=_MESH,
    scratch_types=[
        pltpu.VMEM_SHARED((N_PAD,), jnp.float32),
        pltpu.VMEM((K, CH), jnp.int32),
        pltpu.VMEM((CH,), jnp.float32),
        pltpu.VMEM((CH,), jnp.float32),
        pltpu.SemaphoreType.DMA,
    ],
)
def _deg(col_hbm, zer_hbm, one_hbm, out_hbm, acc, coli, zbuf, obuf, sem):
    c = lax.axis_index("c")
    s = lax.axis_index("s")
    wid = c * 16 + s
    pltpu.sync_copy(zer_hbm, zbuf)
    pltpu.sync_copy(one_hbm, obuf)
    pltpu.sync_copy(col_hbm.at[wid], coli)
    rbase = s * RPS
    for j in range(RPS // CH):
        pltpu.sync_copy(zbuf, acc.at[pl.ds(rbase + j * CH, CH)])
    plsc.subcore_barrier()

    def body(k, carry):
        pltpu.sync_copy(obuf, acc.at[coli.at[k]], add=True)
        return carry

    lax.fori_loop(0, K, body, 0)
    plsc.subcore_barrier()
    for j in range(RPS // CH):
        r0 = rbase + j * CH
        pltpu.sync_copy(acc.at[pl.ds(r0, CH)], zbuf)
        pltpu.sync_copy(zbuf, out_hbm.at[pl.ds(c * N_PAD + r0, CH)])


def _make_prop(FH, NB, KC, feature_split, tc_tiling):
    @functools.partial(
        pl.kernel,
        out_type=jax.ShapeDtypeStruct((2 * N_PAD, FH), jnp.float32),
        mesh=_MESH,
        compiler_params=pltpu.CompilerParams(use_tc_tiling_on_sc=tc_tiling),
        scratch_types=[
            pltpu.VMEM_SHARED((N_PAD, FH), jnp.float32),
            pltpu.VMEM((KC, CH), jnp.int32),
            pltpu.VMEM((KC, CH), jnp.int32),
            [pltpu.VMEM((CH, FH), jnp.float32)] * NB,
            [pltpu.SemaphoreType.DMA] * NB,
            [pltpu.SemaphoreType.DMA] * NB,
        ],
    )
    def prop(g_hbm, row_hbm, col_hbm, zer_hbm, out_hbm, acc, rowi, coli,
             bufs, gs, ss):
        c = lax.axis_index("c")
        s = lax.axis_index("s")
        if feature_split:
            pltpu.sync_copy(row_hbm.at[s], rowi)
            pltpu.sync_copy(col_hbm.at[s], coli)

            def tbody(r, carry):
                for jj in range(CH // 16):
                    v = rowi[r, pl.ds(jj * 16, 16)]
                    rowi[r, pl.ds(jj * 16, 16)] = v * 2 + c
                return carry

            lax.fori_loop(0, KC, tbody, 0)
        else:
            wid = c * 16 + s
            pltpu.sync_copy(row_hbm.at[wid], rowi)
            pltpu.sync_copy(col_hbm.at[wid], coli)
        pltpu.sync_copy(zer_hbm, bufs[0])
        rbase = s * RPS
        for j in range(RPS // CH):
            pltpu.sync_copy(bufs[0], acc.at[pl.ds(rbase + j * CH, CH)])
        plsc.subcore_barrier()

        def gstart(k, u):
            pltpu.async_copy(g_hbm.at[rowi.at[k]], bufs[u], gs[u])

        def gwait(k, u):
            pltpu.make_async_copy(g_hbm.at[rowi.at[k]], bufs[u], gs[u]).wait()

        def sstart(k, u):
            pltpu.async_copy(bufs[u], acc.at[coli.at[k]], ss[u], add=True)

        def swait(k, u):
            pltpu.make_async_copy(bufs[u], acc.at[coli.at[k]], ss[u]).wait()

        for u in range(NB):
            gstart(u, u)

        def body(i, carry):
            k0 = NB * i
            for u in range(NB):
                gwait(k0 + u, u)
                sstart(k0 + u, u)
            for u in range(NB):
                swait(k0 + u, u)
                gstart(k0 + u + NB, u)
            return carry

        lax.fori_loop(0, KC // NB - 1, body, 0)
        t0 = NB * (KC // NB - 1)
        for u in range(NB):
            gwait(t0 + u, u)
            sstart(t0 + u, u)
        for k in range(t0 + NB, KC):
            u = k % NB
            swait(k - NB, u)
            gstart(k, u)
            gwait(k, u)
            sstart(k, u)
        for k in range(max(t0, KC - NB), KC):
            swait(k, k % NB)
        plsc.subcore_barrier()
        for j in range(RPS // CH):
            r0 = rbase + j * CH
            pltpu.sync_copy(acc.at[pl.ds(r0, CH)], bufs[0])
            pltpu.sync_copy(bufs[0], out_hbm.at[pl.ds(c * N_PAD + r0, CH)])

    return prop


_prop128e = _make_prop(IN_CH, 2, K, False, False)
_prop64e = _make_prop(IN_CH // 2, 2, K, False, False)


@functools.partial(
    pl.kernel,
    out_type=jax.ShapeDtypeStruct((NL_PAD,), jnp.float32),
    mesh=_MESH,
    compiler_params=pltpu.CompilerParams(
        use_tc_tiling_on_sc=False, needs_layout_passes=False),
    scratch_types=[
        pltpu.VMEM((KD, CD), jnp.int32),
        pltpu.VMEM((KD, CD), jnp.int32),
        pltpu.VMEM((CD, OUT_CH), jnp.float32),
        pltpu.VMEM((CD, OUT_CH), jnp.float32),
        pltpu.VMEM((CD, OUT_CH), jnp.float32),
        pltpu.VMEM((CD, OUT_CH), jnp.float32),
        pltpu.VMEM((CD, OUT_CH), jnp.float32),
        pltpu.VMEM((CD, OUT_CH), jnp.float32),
        pltpu.VMEM((WPD,), jnp.float32),
        pltpu.SemaphoreType.DMA,
        pltpu.SemaphoreType.DMA,
        pltpu.SemaphoreType.DMA,
        pltpu.SemaphoreType.DMA,
        pltpu.SemaphoreType.DMA,
        pltpu.SemaphoreType.DMA,
    ],
)
def _decode(z_hbm, src_hbm, dst_hbm, out_hbm, srci, dsti, sbuf0, dbuf0,
            sbuf1, dbuf1, sbuf2, dbuf2, obuf, sA0, sB0, sA1, sB1, sA2, sB2):
    c = lax.axis_index("c")
    s = lax.axis_index("s")
    wid = c * 16 + s
    pltpu.sync_copy(src_hbm.at[wid], srci)
    pltpu.sync_copy(dst_hbm.at[wid], dsti)

    sbufs = (sbuf0, sbuf1, sbuf2)
    dbufs = (dbuf0, dbuf1, dbuf2)
    sA = (sA0, sA1, sA2)
    sB = (sB0, sB1, sB2)

    def gstart(k, u):
        pltpu.async_copy(z_hbm.at[srci.at[k]], sbufs[u], sA[u])
        pltpu.async_copy(z_hbm.at[dsti.at[k]], dbufs[u], sB[u])

    def gwait(k, u):
        pltpu.make_async_copy(z_hbm.at[srci.at[k]], sbufs[u], sA[u]).wait()
        pltpu.make_async_copy(z_hbm.at[dsti.at[k]], dbufs[u], sB[u]).wait()

    def compute(k, u):
        sb = sbufs[u]
        db = dbufs[u]
        for t in range(CD // 16):
            rows = lax.iota(jnp.int32, 16) + (16 * t)

            def fb(f4, fc):
                acc, cols = fc
                for _ in range(4):
                    sv = plsc.load_gather(sb, [rows, cols])
                    dv = plsc.load_gather(db, [rows, cols])
                    acc = acc + sv * dv
                    cols = cols + 1
                return (acc, cols)

            acc, _ = lax.fori_loop(
                0, OUT_CH // 4, fb,
                (jnp.zeros((16,), jnp.float32), jnp.zeros((16,), jnp.int32)))
            obuf[pl.ds(k * CD + t * 16, 16)] = acc

    for u in range(3):
        gstart(u, u)

    def body(i, carry):
        k0 = 3 * i
        for u in range(3):
            gwait(k0 + u, u)
            compute(k0 + u, u)
            gstart(k0 + u + 3, u)
        return carry

    lax.fori_loop(0, KD // 3 - 1, body, 0)
    t0 = 3 * (KD // 3 - 1)
    for u in range(3):
        gwait(t0 + u, u)
        compute(t0 + u, u)
        if t0 + u + 3 < KD:
            gstart(t0 + u + 3, u)
    for k in range(t0 + 3, KD):
        gwait(k, k % 3)
        compute(k, k % 3)
    pltpu.sync_copy(obuf, out_hbm.at[pl.ds(wid * WPD, WPD)])


def _tc_pre_body(x_ref, d0_ref, d1_ref, dinv_ref, g0_ref):
    deg = d0_ref[...] + d1_ref[...] + 1.0
    dinv = lax.rsqrt(deg)
    dinv_ref[...] = dinv
    g0_ref[...] = dinv * x_ref[...]


_tc_pre = pl.pallas_call(
    _tc_pre_body,
    grid=(TGRID,),
    in_specs=[
        pl.BlockSpec((BR, IN_CH), lambda i: (i, 0)),
        pl.BlockSpec((BR, 1), lambda i: (i, 0)),
        pl.BlockSpec((BR, 1), lambda i: (i, 0)),
    ],
    out_specs=[
        pl.BlockSpec((BR, 1), lambda i: (i, 0)),
        pl.BlockSpec((BR, IN_CH), lambda i: (i, 0)),
    ],
    out_shape=[
        jax.ShapeDtypeStruct((N_PAD, 1), jnp.float32),
        jax.ShapeDtypeStruct((N_PAD, IN_CH), jnp.float32),
    ],
)


def _tc_mid_body(s0a, s0b, g0, dinv, W10, b10, W20, W11, b11, W21,
                 g2a_ref, g2b_ref):
    dv = dinv[...]
    P = dv * (s0a[...] + s0b[...] + g0[...])
    h0 = jnp.dot(P, W10[...], preferred_element_type=jnp.float32) + b10[...]
    t0 = jnp.dot(h0, W20[...], preferred_element_type=jnp.float32)
    h1 = jnp.dot(P, W11[...], preferred_element_type=jnp.float32) + b11[...]
    t1 = jnp.dot(h1, W21[...], preferred_element_type=jnp.float32)
    g2a_ref[...] = dv * t0
    g2b_ref[...] = dv * t1


def _full(shape):
    return pl.BlockSpec(shape, lambda i: tuple(0 for _ in shape))


_tc_mid = pl.pallas_call(
    _tc_mid_body,
    grid=(TGRID,),
    in_specs=[
        pl.BlockSpec((BR, IN_CH), lambda i: (i, 0)),
        pl.BlockSpec((BR, IN_CH), lambda i: (i, 0)),
        pl.BlockSpec((BR, IN_CH), lambda i: (i, 0)),
        pl.BlockSpec((BR, 1), lambda i: (i, 0)),
        _full((IN_CH, 64)),
        _full((1, 64)),
        _full((64, 64)),
        _full((IN_CH, 128)),
        _full((1, 128)),
        _full((128, 128)),
    ],
    out_specs=[
        pl.BlockSpec((BR, 64), lambda i: (i, 0)),
        pl.BlockSpec((BR, 128), lambda i: (i, 0)),
    ],
    out_shape=[
        jax.ShapeDtypeStruct((N_PAD, 64), jnp.float32),
        jax.ShapeDtypeStruct((N_PAD, 128), jnp.float32),
    ],
)


def _fastkan(carr, ln_g, ln_b, sG, bW, bb):
    mu = jnp.mean(carr, axis=1, keepdims=True)
    xc = carr - mu
    var = jnp.mean(xc * xc, axis=1, keepdims=True)
    xn = xc * lax.rsqrt(var + 1e-5) * ln_g + ln_b
    spline = None
    for g in range(8):
        bas = jnp.exp(-((xn - GRID_PTS[g]) * INV_DENOM) ** 2)
        term = jnp.dot(bas, sG[g], preferred_element_type=jnp.float32)
        spline = term if spline is None else spline + term
    sig = 1.0 / (1.0 + jnp.exp(-carr))
    base = jnp.dot(carr * sig, bW, preferred_element_type=jnp.float32) + bb
    return spline + base


def _tc_post_body(s2a0, s2a1, s2b0, s2b1, g2a, g2b, dinv, b20, b21,
                  lng0, lnb0, lng1, lnb1, sG0, sG1, bW0, bW1, bb0, bb1, att,
                  z_ref):
    dv = dinv[...]
    c0 = dv * (s2a0[...] + s2a1[...] + g2a[...]) + b20[...]
    c1 = dv * (s2b0[...] + s2b1[...] + g2b[...]) + b21[...]
    out0 = _fastkan(c0, lng0[...], lnb0[...], sG0, bW0[...], bb0[...])
    out1 = _fastkan(c1, lng1[...], lnb1[...], sG1, bW1[...], bb1[...])
    a0 = att[0]
    a1 = att[1]
    m = jnp.maximum(a0, a1)
    e0 = jnp.exp(jnp.full((BR, OUT_CH), a0 - m, jnp.float32))
    e1 = jnp.exp(jnp.full((BR, OUT_CH), a1 - m, jnp.float32))
    z_ref[...] = (e0 * out0 + e1 * out1) / (e0 + e1)


_tc_post = pl.pallas_call(
    _tc_post_body,
    grid=(TGRID,),
    in_specs=[
        pl.BlockSpec((BR, 64), lambda i: (i, 0)),
        pl.BlockSpec((BR, 64), lambda i: (i, 0)),
        pl.BlockSpec((BR, 128), lambda i: (i, 0)),
        pl.BlockSpec((BR, 128), lambda i: (i, 0)),
        pl.BlockSpec((BR, 64), lambda i: (i, 0)),
        pl.BlockSpec((BR, 128), lambda i: (i, 0)),
        pl.BlockSpec((BR, 1), lambda i: (i, 0)),
        _full((1, 64)),
        _full((1, 128)),
        _full((1, 64)),
        _full((1, 64)),
        _full((1, 128)),
        _full((1, 128)),
        _full((8, 64, 64)),
        _full((8, 128, 64)),
        _full((64, 64)),
        _full((128, 64)),
        _full((1, 64)),
        _full((1, 64)),
        pl.BlockSpec(memory_space=pltpu.SMEM),
    ],
    out_specs=pl.BlockSpec((BR, OUT_CH), lambda i: (i, 0)),
    out_shape=jax.ShapeDtypeStruct((N_PAD, OUT_CH), jnp.float32),
)


def kernel(x, edge_index, edge_label_index, att_weights,
           p0_W1, p0_b1, p0_W2, p0_b2, p0_ln_g, p0_ln_b, p0_spline_W,
           p0_base_W, p0_base_b,
           p1_W1, p1_b1, p1_W2, p1_b2, p1_ln_g, p1_ln_b, p1_spline_W,
           p1_base_W, p1_base_b):
    f32 = jnp.float32
    x_pad = jnp.concatenate([x, jnp.zeros((N_PAD - N, IN_CH), f32)], axis=0)
    row3 = edge_index[0].reshape(NW, K, CH)
    col3 = edge_index[1].reshape(NW, K, CH)
    zCH = jnp.zeros((CH,), f32)
    oCH = jnp.ones((CH,), f32)
    z128 = jnp.zeros((CH, IN_CH), f32)
    z64 = jnp.zeros((CH, IN_CH // 2), f32)

    degp = _deg(col3, zCH, oCH)
    d0 = degp[:N_PAD].reshape(N_PAD, 1)
    d1 = degp[N_PAD:].reshape(N_PAD, 1)
    dinv, g0 = _tc_pre(x_pad, d0, d1)

    s0 = _prop128e(g0, row3, col3, z128)
    g2a, g2b = _tc_mid(s0[:N_PAD], s0[N_PAD:], g0, dinv,
                       p0_W1, p0_b1.reshape(1, 64), p0_W2,
                       p1_W1, p1_b1.reshape(1, 128), p1_W2)

    s2a = _prop64e(g2a, row3, col3, z64)
    s2b = _prop128e(g2b, row3, col3, z128)
    sG0 = p0_spline_W.reshape(64, 8, 64).transpose(1, 0, 2)
    sG1 = p1_spline_W.reshape(128, 8, 64).transpose(1, 0, 2)
    z = _tc_post(s2a[:N_PAD], s2a[N_PAD:], s2b[:N_PAD], s2b[N_PAD:],
                 g2a, g2b, dinv,
                 p0_b2.reshape(1, 64), p1_b2.reshape(1, 128),
                 p0_ln_g.reshape(1, 64), p0_ln_b.reshape(1, 64),
                 p1_ln_g.reshape(1, 128), p1_ln_b.reshape(1, 128),
                 sG0, sG1, p0_base_W, p1_base_W,
                 p0_base_b.reshape(1, 64), p1_base_b.reshape(1, 64),
                 att_weights)

    pad = NL_PAD - NL
    padidx = (jnp.arange(pad, dtype=jnp.int32) * 13) % N
    src3 = jnp.concatenate([edge_label_index[0], padidx]).reshape(NW, KD, CD)
    dst3 = jnp.concatenate([edge_label_index[1], padidx]).reshape(NW, KD, CD)
    res = _decode(z, src3, dst3)
    return res[:NL]


# speedup vs baseline: 1.0584x; 1.0584x over previous
import functools

import jax
import jax.numpy as jnp
from jax import lax
from jax.experimental import pallas as pl
from jax.experimental.pallas import tpu as pltpu
from jax.experimental.pallas import tpu_sc as plsc

N = 10000
N_PAD = 10240
E = 320000
NL = 200000
IN_CH = 128
F2 = 192
OUT_CH = 64

NW = 32
EPW = E // NW
CH = 80
K = EPW // CH
EPT = E // 16
K2 = EPT // CH
RPS = N_PAD // 16
F2H = F2 // 2

CD = 64
WPD = 6272
KD = WPD // CD
NL_PAD = NW * WPD

BR = 512
TGRID = N_PAD // BR

_MESH = plsc.VectorSubcoreMesh(core_axis_name="c", subcore_axis_name="s")

GRID_PTS = [-2.0 + j * (4.0 / 7.0) for j in range(8)]
INV_DENOM = 7.0 / 4.0


@functools.partial(
    pl.kernel,
    out_type=jax.ShapeDtypeStruct((2 * N_PAD,), jnp.float32),
    mesh=_MESH,
    scratch_types=[
        pltpu.VMEM_SHARED((N_PAD,), jnp.float32),
        pltpu.VMEM((K, CH), jnp.int32),
        pltpu.VMEM((CH,), jnp.float32),
        pltpu.VMEM((CH,), jnp.float32),
        pltpu.SemaphoreType.DMA,
    ],
)
def _deg(col_hbm, zer_hbm, one_hbm, out_hbm, acc, coli, zbuf, obuf, sem):
    c = lax.axis_index("c")
    s = lax.axis_index("s")
    wid = c * 16 + s
    pltpu.sync_copy(zer_hbm, zbuf)
    pltpu.sync_copy(one_hbm, obuf)
    pltpu.sync_copy(col_hbm.at[wid], coli)
    rbase = s * RPS
    for j in range(RPS // CH):
        pltpu.sync_copy(zbuf, acc.at[pl.ds(rbase + j * CH, CH)])
    plsc.subcore_barrier()

    def body(k, carry):
        pltpu.sync_copy(obuf, acc.at[coli.at[k]], add=True)
        return carry

    lax.fori_loop(0, K, body, 0)
    plsc.subcore_barrier()
    for j in range(RPS // CH):
        r0 = rbase + j * CH
        pltpu.sync_copy(acc.at[pl.ds(r0, CH)], zbuf)
        pltpu.sync_copy(zbuf, out_hbm.at[pl.ds(c * N_PAD + r0, CH)])


def _make_prop(FH, NB, KC, feature_split, tc_tiling):
    @functools.partial(
        pl.kernel,
        out_type=jax.ShapeDtypeStruct((2 * N_PAD, FH), jnp.float32),
        mesh=_MESH,
        compiler_params=pltpu.CompilerParams(use_tc_tiling_on_sc=tc_tiling),
        scratch_types=[
            pltpu.VMEM_SHARED((N_PAD, FH), jnp.float32),
            pltpu.VMEM((KC, CH), jnp.int32),
            pltpu.VMEM((KC, CH), jnp.int32),
            [pltpu.VMEM((CH, FH), jnp.float32)] * NB,
            [pltpu.SemaphoreType.DMA] * NB,
            [pltpu.SemaphoreType.DMA] * NB,
        ],
    )
    def prop(g_hbm, row_hbm, col_hbm, zer_hbm, out_hbm, acc, rowi, coli,
             bufs, gs, ss):
        c = lax.axis_index("c")
        s = lax.axis_index("s")
        if feature_split:
            pltpu.sync_copy(row_hbm.at[s], rowi)
            pltpu.sync_copy(col_hbm.at[s], coli)

            def tbody(r, carry):
                for jj in range(CH // 16):
                    v = rowi[r, pl.ds(jj * 16, 16)]
                    rowi[r, pl.ds(jj * 16, 16)] = v * 2 + c
                return carry

            lax.fori_loop(0, KC, tbody, 0)
        else:
            wid = c * 16 + s
            pltpu.sync_copy(row_hbm.at[wid], rowi)
            pltpu.sync_copy(col_hbm.at[wid], coli)
        pltpu.sync_copy(zer_hbm, bufs[0])
        rbase = s * RPS
        for j in range(RPS // CH):
            pltpu.sync_copy(bufs[0], acc.at[pl.ds(rbase + j * CH, CH)])
        plsc.subcore_barrier()

        def gstart(k, u):
            pltpu.async_copy(g_hbm.at[rowi.at[k]], bufs[u], gs[u])

        def gwait(k, u):
            pltpu.make_async_copy(g_hbm.at[rowi.at[k]], bufs[u], gs[u]).wait()

        def sstart(k, u):
            pltpu.async_copy(bufs[u], acc.at[coli.at[k]], ss[u], add=True)

        def swait(k, u):
            pltpu.make_async_copy(bufs[u], acc.at[coli.at[k]], ss[u]).wait()

        for u in range(NB):
            gstart(u, u)

        def body(i, carry):
            k0 = NB * i
            for u in range(NB):
                gwait(k0 + u, u)
                sstart(k0 + u, u)
            for u in range(NB):
                swait(k0 + u, u)
                gstart(k0 + u + NB, u)
            return carry

        lax.fori_loop(0, KC // NB - 1, body, 0)
        t0 = NB * (KC // NB - 1)
        for u in range(NB):
            gwait(t0 + u, u)
            sstart(t0 + u, u)
        for k in range(t0 + NB, KC):
            u = k % NB
            swait(k - NB, u)
            gstart(k, u)
            gwait(k, u)
            sstart(k, u)
        for k in range(max(t0, KC - NB), KC):
            swait(k, k % NB)
        plsc.subcore_barrier()
        for j in range(RPS // CH):
            r0 = rbase + j * CH
            pltpu.sync_copy(acc.at[pl.ds(r0, CH)], bufs[0])
            pltpu.sync_copy(bufs[0], out_hbm.at[pl.ds(c * N_PAD + r0, CH)])

    return prop


_prop128e = _make_prop(IN_CH, 2, K, False, False)
_prop96 = _make_prop(F2H, 3, K2, True, False)


@functools.partial(
    pl.kernel,
    out_type=jax.ShapeDtypeStruct((NL_PAD,), jnp.float32),
    mesh=_MESH,
    compiler_params=pltpu.CompilerParams(
        use_tc_tiling_on_sc=False, needs_layout_passes=False),
    scratch_types=[
        pltpu.VMEM((KD, CD), jnp.int32),
        pltpu.VMEM((KD, CD), jnp.int32),
        pltpu.VMEM((CD, OUT_CH), jnp.float32),
        pltpu.VMEM((CD, OUT_CH), jnp.float32),
        pltpu.VMEM((CD, OUT_CH), jnp.float32),
        pltpu.VMEM((CD, OUT_CH), jnp.float32),
        pltpu.VMEM((CD, OUT_CH), jnp.float32),
        pltpu.VMEM((CD, OUT_CH), jnp.float32),
        pltpu.VMEM((WPD,), jnp.float32),
        pltpu.SemaphoreType.DMA,
        pltpu.SemaphoreType.DMA,
        pltpu.SemaphoreType.DMA,
        pltpu.SemaphoreType.DMA,
        pltpu.SemaphoreType.DMA,
        pltpu.SemaphoreType.DMA,
    ],
)
def _decode(z_hbm, src_hbm, dst_hbm, out_hbm, srci, dsti, sbuf0, dbuf0,
            sbuf1, dbuf1, sbuf2, dbuf2, obuf, sA0, sB0, sA1, sB1, sA2, sB2):
    c = lax.axis_index("c")
    s = lax.axis_index("s")
    wid = c * 16 + s
    pltpu.sync_copy(src_hbm.at[wid], srci)
    pltpu.sync_copy(dst_hbm.at[wid], dsti)

    sbufs = (sbuf0, sbuf1, sbuf2)
    dbufs = (dbuf0, dbuf1, dbuf2)
    sA = (sA0, sA1, sA2)
    sB = (sB0, sB1, sB2)

    def gstart(k, u):
        pltpu.async_copy(z_hbm.at[srci.at[k]], sbufs[u], sA[u])
        pltpu.async_copy(z_hbm.at[dsti.at[k]], dbufs[u], sB[u])

    def gwait(k, u):
        pltpu.make_async_copy(z_hbm.at[srci.at[k]], sbufs[u], sA[u]).wait()
        pltpu.make_async_copy(z_hbm.at[dsti.at[k]], dbufs[u], sB[u]).wait()

    zero16 = jnp.zeros((16,), jnp.int32)

    def compute(k, u):
        sb = sbufs[u]
        db = dbufs[u]
        for t in range(CD // 16):
            base = (lax.iota(jnp.int32, 16) + 16 * t) * OUT_CH

            def fb(f4, fc):
                acc, cols = fc
                for _ in range(4):
                    sv = plsc.load_gather(sb, [zero16, cols])
                    dv = plsc.load_gather(db, [zero16, cols])
                    acc = acc + sv * dv
                    cols = cols + 1
                return (acc, cols)

            acc, _ = lax.fori_loop(
                0, OUT_CH // 4, fb,
                (jnp.zeros((16,), jnp.float32), base))
            obuf[pl.ds(k * CD + t * 16, 16)] = acc

    for u in range(3):
        gstart(u, u)

    def body(i, carry):
        k0 = 3 * i
        for u in range(3):
            gwait(k0 + u, u)
            compute(k0 + u, u)
            gstart(k0 + u + 3, u)
        return carry

    lax.fori_loop(0, KD // 3 - 1, body, 0)
    t0 = 3 * (KD // 3 - 1)
    for u in range(3):
        gwait(t0 + u, u)
        compute(t0 + u, u)
        if t0 + u + 3 < KD:
            gstart(t0 + u + 3, u)
    for k in range(t0 + 3, KD):
        gwait(k, k % 3)
        compute(k, k % 3)
    pltpu.sync_copy(obuf, out_hbm.at[pl.ds(wid * WPD, WPD)])


def _tc_pre_body(x_ref, d0_ref, d1_ref, dinv_ref, g0_ref):
    deg = d0_ref[...] + d1_ref[...] + 1.0
    dinv = lax.rsqrt(deg)
    dinv_ref[...] = dinv
    g0_ref[...] = dinv * x_ref[...]


_tc_pre = pl.pallas_call(
    _tc_pre_body,
    grid=(TGRID,),
    in_specs=[
        pl.BlockSpec((BR, IN_CH), lambda i: (i, 0)),
        pl.BlockSpec((BR, 1), lambda i: (i, 0)),
        pl.BlockSpec((BR, 1), lambda i: (i, 0)),
    ],
    out_specs=[
        pl.BlockSpec((BR, 1), lambda i: (i, 0)),
        pl.BlockSpec((BR, IN_CH), lambda i: (i, 0)),
    ],
    out_shape=[
        jax.ShapeDtypeStruct((N_PAD, 1), jnp.float32),
        jax.ShapeDtypeStruct((N_PAD, IN_CH), jnp.float32),
    ],
)


def _tc_mid_body(s0a, s0b, g0, dinv, W10, b10, W20, W11, b11, W21, g2_ref):
    dv = dinv[...]
    P = dv * (s0a[...] + s0b[...] + g0[...])
    h0 = jnp.dot(P, W10[...], preferred_element_type=jnp.float32) + b10[...]
    t0 = jnp.dot(h0, W20[...], preferred_element_type=jnp.float32)
    h1 = jnp.dot(P, W11[...], preferred_element_type=jnp.float32) + b11[...]
    t1 = jnp.dot(h1, W21[...], preferred_element_type=jnp.float32)
    g2_ref[...] = jnp.concatenate([dv * t0, dv * t1], axis=1)


def _full(shape):
    return pl.BlockSpec(shape, lambda i: tuple(0 for _ in shape))


_tc_mid = pl.pallas_call(
    _tc_mid_body,
    grid=(TGRID,),
    in_specs=[
        pl.BlockSpec((BR, IN_CH), lambda i: (i, 0)),
        pl.BlockSpec((BR, IN_CH), lambda i: (i, 0)),
        pl.BlockSpec((BR, IN_CH), lambda i: (i, 0)),
        pl.BlockSpec((BR, 1), lambda i: (i, 0)),
        _full((IN_CH, 64)),
        _full((1, 64)),
        _full((64, 64)),
        _full((IN_CH, 128)),
        _full((1, 128)),
        _full((128, 128)),
    ],
    out_specs=pl.BlockSpec((BR, F2), lambda i: (i, 0)),
    out_shape=jax.ShapeDtypeStruct((N_PAD, F2), jnp.float32),
)


def _fastkan(carr, ln_g, ln_b, sG, bW, bb):
    mu = jnp.mean(carr, axis=1, keepdims=True)
    xc = carr - mu
    var = jnp.mean(xc * xc, axis=1, keepdims=True)
    xn = xc * lax.rsqrt(var + 1e-5) * ln_g + ln_b
    spline = None
    for g in range(8):
        bas = jnp.exp(-((xn - GRID_PTS[g]) * INV_DENOM) ** 2)
        term = jnp.dot(bas, sG[g], preferred_element_type=jnp.float32)
        spline = term if spline is None else spline + term
    sig = 1.0 / (1.0 + jnp.exp(-carr))
    base = jnp.dot(carr * sig, bW, preferred_element_type=jnp.float32) + bb
    return spline + base


def _tc_post_body(s2a, s2b, g2, dinv, b20, b21,
                  lng0, lnb0, lng1, lnb1, sG0, sG1, bW0, bW1, bb0, bb1, att,
                  z_ref):
    dv = dinv[...]
    s2 = jnp.concatenate([s2a[...], s2b[...]], axis=1)
    c0 = dv * (s2[:, :64] + g2[:, :64]) + b20[...]
    c1 = dv * (s2[:, 64:] + g2[:, 64:]) + b21[...]
    out0 = _fastkan(c0, lng0[...], lnb0[...], sG0, bW0[...], bb0[...])
    out1 = _fastkan(c1, lng1[...], lnb1[...], sG1, bW1[...], bb1[...])
    a0 = att[0]
    a1 = att[1]
    m = jnp.maximum(a0, a1)
    e0 = jnp.exp(jnp.full((BR, OUT_CH), a0 - m, jnp.float32))
    e1 = jnp.exp(jnp.full((BR, OUT_CH), a1 - m, jnp.float32))
    z_ref[...] = (e0 * out0 + e1 * out1) / (e0 + e1)


_tc_post = pl.pallas_call(
    _tc_post_body,
    grid=(TGRID,),
    in_specs=[
        pl.BlockSpec((BR, F2H), lambda i: (i, 0)),
        pl.BlockSpec((BR, F2H), lambda i: (i, 0)),
        pl.BlockSpec((BR, F2), lambda i: (i, 0)),
        pl.BlockSpec((BR, 1), lambda i: (i, 0)),
        _full((1, 64)),
        _full((1, 128)),
        _full((1, 64)),
        _full((1, 64)),
        _full((1, 128)),
        _full((1, 128)),
        _full((8, 64, 64)),
        _full((8, 128, 64)),
        _full((64, 64)),
        _full((128, 64)),
        _full((1, 64)),
        _full((1, 64)),
        pl.BlockSpec(memory_space=pltpu.SMEM),
    ],
    out_specs=pl.BlockSpec((BR, OUT_CH), lambda i: (i, 0)),
    out_shape=jax.ShapeDtypeStruct((N_PAD, OUT_CH), jnp.float32),
)


def kernel(x, edge_index, edge_label_index, att_weights,
           p0_W1, p0_b1, p0_W2, p0_b2, p0_ln_g, p0_ln_b, p0_spline_W,
           p0_base_W, p0_base_b,
           p1_W1, p1_b1, p1_W2, p1_b2, p1_ln_g, p1_ln_b, p1_spline_W,
           p1_base_W, p1_base_b):
    f32 = jnp.float32
    x_pad = jnp.concatenate([x, jnp.zeros((N_PAD - N, IN_CH), f32)], axis=0)
    row3 = edge_index[0].reshape(NW, K, CH)
    col3 = edge_index[1].reshape(NW, K, CH)
    row3b = edge_index[0].reshape(16, K2, CH)
    col3b = edge_index[1].reshape(16, K2, CH)
    zCH = jnp.zeros((CH,), f32)
    oCH = jnp.ones((CH,), f32)
    z128 = jnp.zeros((CH, IN_CH), f32)
    z96 = jnp.zeros((CH, F2H), f32)

    degp = _deg(col3, zCH, oCH)
    d0 = degp[:N_PAD].reshape(N_PAD, 1)
    d1 = degp[N_PAD:].reshape(N_PAD, 1)
    dinv, g0 = _tc_pre(x_pad, d0, d1)

    s0 = _prop128e(g0, row3, col3, z128)
    g2 = _tc_mid(s0[:N_PAD], s0[N_PAD:], g0, dinv,
                 p0_W1, p0_b1.reshape(1, 64), p0_W2,
                 p1_W1, p1_b1.reshape(1, 128), p1_W2)

    g2v = g2.reshape(2 * N_PAD, F2H)
    s2 = _prop96(g2v, row3b, col3b, z96)
    sG0 = p0_spline_W.reshape(64, 8, 64).transpose(1, 0, 2)
    sG1 = p1_spline_W.reshape(128, 8, 64).transpose(1, 0, 2)
    z = _tc_post(s2[:N_PAD], s2[N_PAD:], g2, dinv,
                 p0_b2.reshape(1, 64), p1_b2.reshape(1, 128),
                 p0_ln_g.reshape(1, 64), p0_ln_b.reshape(1, 64),
                 p1_ln_g.reshape(1, 128), p1_ln_b.reshape(1, 128),
                 sG0, sG1, p0_base_W, p1_base_W,
                 p0_base_b.reshape(1, 64), p1_base_b.reshape(1, 64),
                 att_weights)

    pad = NL_PAD - NL
    padidx = (jnp.arange(pad, dtype=jnp.int32) * 13) % N
    src3 = jnp.concatenate([edge_label_index[0], padidx]).reshape(NW, KD, CD)
    dst3 = jnp.concatenate([edge_label_index[1], padidx]).reshape(NW, KD, CD)
    res = _decode(z, src3, dst3)
    return res[:NL]


# speedup vs baseline: 1.5867x; 1.4992x over previous
import functools

import jax
import jax.numpy as jnp
from jax import lax
from jax.experimental import pallas as pl
from jax.experimental.pallas import tpu as pltpu
from jax.experimental.pallas import tpu_sc as plsc

N = 10000
N_PAD = 10240
E = 320000
NL = 200000
IN_CH = 128
F2 = 192
OUT_CH = 64

NW = 32
EPW = E // NW
CH = 80
K = EPW // CH
EPT = E // 16
K2 = EPT // CH
RPS = N_PAD // 16
F2H = F2 // 2

CD = 64
WPD = 6272
KD = WPD // CD
NL_PAD = NW * WPD

BR = 512
TGRID = N_PAD // BR

_MESH = plsc.VectorSubcoreMesh(core_axis_name="c", subcore_axis_name="s")

GRID_PTS = [-2.0 + j * (4.0 / 7.0) for j in range(8)]
INV_DENOM = 7.0 / 4.0


@functools.partial(
    pl.kernel,
    out_type=jax.ShapeDtypeStruct((2 * N_PAD,), jnp.float32),
    mesh=_MESH,
    scratch_types=[
        pltpu.VMEM_SHARED((N_PAD,), jnp.float32),
        pltpu.VMEM((K, CH), jnp.int32),
        pltpu.VMEM((CH,), jnp.float32),
        pltpu.VMEM((CH,), jnp.float32),
        pltpu.SemaphoreType.DMA,
    ],
)
def _deg(col_hbm, zer_hbm, one_hbm, out_hbm, acc, coli, zbuf, obuf, sem):
    c = lax.axis_index("c")
    s = lax.axis_index("s")
    wid = c * 16 + s
    pltpu.sync_copy(zer_hbm, zbuf)
    pltpu.sync_copy(one_hbm, obuf)
    pltpu.sync_copy(col_hbm.at[wid], coli)
    rbase = s * RPS
    for j in range(RPS // CH):
        pltpu.sync_copy(zbuf, acc.at[pl.ds(rbase + j * CH, CH)])
    plsc.subcore_barrier()

    def body(k, carry):
        pltpu.sync_copy(obuf, acc.at[coli.at[k]], add=True)
        return carry

    lax.fori_loop(0, K, body, 0)
    plsc.subcore_barrier()
    for j in range(RPS // CH):
        r0 = rbase + j * CH
        pltpu.sync_copy(acc.at[pl.ds(r0, CH)], zbuf)
        pltpu.sync_copy(zbuf, out_hbm.at[pl.ds(c * N_PAD + r0, CH)])


def _make_prop(FH, NB, KC, feature_split, tc_tiling):
    @functools.partial(
        pl.kernel,
        out_type=jax.ShapeDtypeStruct((2 * N_PAD, FH), jnp.float32),
        mesh=_MESH,
        compiler_params=pltpu.CompilerParams(use_tc_tiling_on_sc=tc_tiling),
        scratch_types=[
            pltpu.VMEM_SHARED((N_PAD, FH), jnp.float32),
            pltpu.VMEM((KC, CH), jnp.int32),
            pltpu.VMEM((KC, CH), jnp.int32),
            [pltpu.VMEM((CH, FH), jnp.float32)] * NB,
            [pltpu.SemaphoreType.DMA] * NB,
            [pltpu.SemaphoreType.DMA] * NB,
        ],
    )
    def prop(g_hbm, row_hbm, col_hbm, zer_hbm, out_hbm, acc, rowi, coli,
             bufs, gs, ss):
        c = lax.axis_index("c")
        s = lax.axis_index("s")
        if feature_split:
            pltpu.sync_copy(row_hbm.at[s], rowi)
            pltpu.sync_copy(col_hbm.at[s], coli)

            def tbody(r, carry):
                for jj in range(CH // 16):
                    v = rowi[r, pl.ds(jj * 16, 16)]
                    rowi[r, pl.ds(jj * 16, 16)] = v * 2 + c
                return carry

            lax.fori_loop(0, KC, tbody, 0)
        else:
            wid = c * 16 + s
            pltpu.sync_copy(row_hbm.at[wid], rowi)
            pltpu.sync_copy(col_hbm.at[wid], coli)
        pltpu.sync_copy(zer_hbm, bufs[0])
        rbase = s * RPS
        for j in range(RPS // CH):
            pltpu.sync_copy(bufs[0], acc.at[pl.ds(rbase + j * CH, CH)])
        plsc.subcore_barrier()

        def gstart(k, u):
            pltpu.async_copy(g_hbm.at[rowi.at[k]], bufs[u], gs[u])

        def gwait(k, u):
            pltpu.make_async_copy(g_hbm.at[rowi.at[k]], bufs[u], gs[u]).wait()

        def sstart(k, u):
            pltpu.async_copy(bufs[u], acc.at[coli.at[k]], ss[u], add=True)

        def swait(k, u):
            pltpu.make_async_copy(bufs[u], acc.at[coli.at[k]], ss[u]).wait()

        for u in range(NB):
            gstart(u, u)

        def body(i, carry):
            k0 = NB * i
            for u in range(NB):
                gwait(k0 + u, u)
                sstart(k0 + u, u)
            for u in range(NB):
                swait(k0 + u, u)
                gstart(k0 + u + NB, u)
            return carry

        lax.fori_loop(0, KC // NB - 1, body, 0)
        t0 = NB * (KC // NB - 1)
        for u in range(NB):
            gwait(t0 + u, u)
            sstart(t0 + u, u)
        for k in range(t0 + NB, KC):
            u = k % NB
            swait(k - NB, u)
            gstart(k, u)
            gwait(k, u)
            sstart(k, u)
        for k in range(max(t0, KC - NB), KC):
            swait(k, k % NB)
        plsc.subcore_barrier()
        for j in range(RPS // CH):
            r0 = rbase + j * CH
            pltpu.sync_copy(acc.at[pl.ds(r0, CH)], bufs[0])
            pltpu.sync_copy(bufs[0], out_hbm.at[pl.ds(c * N_PAD + r0, CH)])

    return prop


_prop128e = _make_prop(IN_CH, 2, K, False, False)
_prop96 = _make_prop(F2H, 3, K2, True, False)


@functools.partial(
    pl.kernel,
    out_type=jax.ShapeDtypeStruct((NL_PAD,), jnp.float32),
    mesh=_MESH,
    compiler_params=pltpu.CompilerParams(
        use_tc_tiling_on_sc=False, needs_layout_passes=False),
    scratch_types=[
        pltpu.VMEM((KD, CD), jnp.int32),
        pltpu.VMEM((KD, CD), jnp.int32),
        pltpu.VMEM((CD, OUT_CH), jnp.float32),
        pltpu.VMEM((CD, OUT_CH), jnp.float32),
        pltpu.VMEM((CD, OUT_CH), jnp.float32),
        pltpu.VMEM((CD, OUT_CH), jnp.float32),
        pltpu.VMEM((CD, OUT_CH), jnp.float32),
        pltpu.VMEM((CD, OUT_CH), jnp.float32),
        pltpu.VMEM((WPD,), jnp.float32),
        pltpu.SemaphoreType.DMA,
        pltpu.SemaphoreType.DMA,
        pltpu.SemaphoreType.DMA,
        pltpu.SemaphoreType.DMA,
        pltpu.SemaphoreType.DMA,
        pltpu.SemaphoreType.DMA,
    ],
)
def _decode(z_hbm, src_hbm, dst_hbm, out_hbm, srci, dsti, sbuf0, dbuf0,
            sbuf1, dbuf1, sbuf2, dbuf2, obuf, sA0, sB0, sA1, sB1, sA2, sB2):
    c = lax.axis_index("c")
    s = lax.axis_index("s")
    wid = c * 16 + s
    pltpu.sync_copy(src_hbm.at[wid], srci)
    pltpu.sync_copy(dst_hbm.at[wid], dsti)

    sbufs = (sbuf0, sbuf1, sbuf2)
    dbufs = (dbuf0, dbuf1, dbuf2)
    sA = (sA0, sA1, sA2)
    sB = (sB0, sB1, sB2)

    def gstart(k, u):
        pltpu.async_copy(z_hbm.at[srci.at[k]], sbufs[u], sA[u])
        pltpu.async_copy(z_hbm.at[dsti.at[k]], dbufs[u], sB[u])

    def gwait(k, u):
        pltpu.make_async_copy(z_hbm.at[srci.at[k]], sbufs[u], sA[u]).wait()
        pltpu.make_async_copy(z_hbm.at[dsti.at[k]], dbufs[u], sB[u]).wait()

    def compute(k, u):
        sb = sbufs[u]
        db = dbufs[u]

        def gb(g16, carry2):
            acc = jnp.zeros((16,), jnp.float32)
            for w in range(16):
                e = g16 * 16 + w
                p = None
                for j in range(OUT_CH // 16):
                    sv = sb[e, pl.ds(j * 16, 16)]
                    dv = db[e, pl.ds(j * 16, 16)]
                    p = sv * dv if p is None else p + sv * dv
                r = jnp.sum(p)
                acc = jnp.where(lax.iota(jnp.int32, 16) == w, r, acc)
            obuf[pl.ds(k * CD + g16 * 16, 16)] = acc
            return carry2

        lax.fori_loop(0, CD // 16, gb, 0)

    for u in range(3):
        gstart(u, u)

    def body(i, carry):
        k0 = 3 * i
        for u in range(3):
            gwait(k0 + u, u)
            compute(k0 + u, u)
            gstart(k0 + u + 3, u)
        return carry

    lax.fori_loop(0, KD // 3 - 1, body, 0)
    t0 = 3 * (KD // 3 - 1)
    for u in range(3):
        gwait(t0 + u, u)
        compute(t0 + u, u)
        if t0 + u + 3 < KD:
            gstart(t0 + u + 3, u)
    for k in range(t0 + 3, KD):
        gwait(k, k % 3)
        compute(k, k % 3)
    pltpu.sync_copy(obuf, out_hbm.at[pl.ds(wid * WPD, WPD)])


def _tc_pre_body(x_ref, d0_ref, d1_ref, dinv_ref, g0_ref):
    deg = d0_ref[...] + d1_ref[...] + 1.0
    dinv = lax.rsqrt(deg)
    dinv_ref[...] = dinv
    g0_ref[...] = dinv * x_ref[...]


_tc_pre = pl.pallas_call(
    _tc_pre_body,
    grid=(TGRID,),
    in_specs=[
        pl.BlockSpec((BR, IN_CH), lambda i: (i, 0)),
        pl.BlockSpec((BR, 1), lambda i: (i, 0)),
        pl.BlockSpec((BR, 1), lambda i: (i, 0)),
    ],
    out_specs=[
        pl.BlockSpec((BR, 1), lambda i: (i, 0)),
        pl.BlockSpec((BR, IN_CH), lambda i: (i, 0)),
    ],
    out_shape=[
        jax.ShapeDtypeStruct((N_PAD, 1), jnp.float32),
        jax.ShapeDtypeStruct((N_PAD, IN_CH), jnp.float32),
    ],
)


def _tc_mid_body(s0a, s0b, g0, dinv, W10, b10, W20, W11, b11, W21, g2_ref):
    dv = dinv[...]
    P = dv * (s0a[...] + s0b[...] + g0[...])
    h0 = jnp.dot(P, W10[...], preferred_element_type=jnp.float32) + b10[...]
    t0 = jnp.dot(h0, W20[...], preferred_element_type=jnp.float32)
    h1 = jnp.dot(P, W11[...], preferred_element_type=jnp.float32) + b11[...]
    t1 = jnp.dot(h1, W21[...], preferred_element_type=jnp.float32)
    g2_ref[...] = jnp.concatenate([dv * t0, dv * t1], axis=1)


def _full(shape):
    return pl.BlockSpec(shape, lambda i: tuple(0 for _ in shape))


_tc_mid = pl.pallas_call(
    _tc_mid_body,
    grid=(TGRID,),
    in_specs=[
        pl.BlockSpec((BR, IN_CH), lambda i: (i, 0)),
        pl.BlockSpec((BR, IN_CH), lambda i: (i, 0)),
        pl.BlockSpec((BR, IN_CH), lambda i: (i, 0)),
        pl.BlockSpec((BR, 1), lambda i: (i, 0)),
        _full((IN_CH, 64)),
        _full((1, 64)),
        _full((64, 64)),
        _full((IN_CH, 128)),
        _full((1, 128)),
        _full((128, 128)),
    ],
    out_specs=pl.BlockSpec((BR, F2), lambda i: (i, 0)),
    out_shape=jax.ShapeDtypeStruct((N_PAD, F2), jnp.float32),
)


def _fastkan(carr, ln_g, ln_b, sG, bW, bb):
    mu = jnp.mean(carr, axis=1, keepdims=True)
    xc = carr - mu
    var = jnp.mean(xc * xc, axis=1, keepdims=True)
    xn = xc * lax.rsqrt(var + 1e-5) * ln_g + ln_b
    spline = None
    for g in range(8):
        bas = jnp.exp(-((xn - GRID_PTS[g]) * INV_DENOM) ** 2)
        term = jnp.dot(bas, sG[g], preferred_element_type=jnp.float32)
        spline = term if spline is None else spline + term
    sig = 1.0 / (1.0 + jnp.exp(-carr))
    base = jnp.dot(carr * sig, bW, preferred_element_type=jnp.float32) + bb
    return spline + base


def _tc_post_body(s2a, s2b, g2, dinv, b20, b21,
                  lng0, lnb0, lng1, lnb1, sG0, sG1, bW0, bW1, bb0, bb1, att,
                  z_ref):
    dv = dinv[...]
    s2 = jnp.concatenate([s2a[...], s2b[...]], axis=1)
    c0 = dv * (s2[:, :64] + g2[:, :64]) + b20[...]
    c1 = dv * (s2[:, 64:] + g2[:, 64:]) + b21[...]
    out0 = _fastkan(c0, lng0[...], lnb0[...], sG0, bW0[...], bb0[...])
    out1 = _fastkan(c1, lng1[...], lnb1[...], sG1, bW1[...], bb1[...])
    a0 = att[0]
    a1 = att[1]
    m = jnp.maximum(a0, a1)
    e0 = jnp.exp(jnp.full((BR, OUT_CH), a0 - m, jnp.float32))
    e1 = jnp.exp(jnp.full((BR, OUT_CH), a1 - m, jnp.float32))
    z_ref[...] = (e0 * out0 + e1 * out1) / (e0 + e1)


_tc_post = pl.pallas_call(
    _tc_post_body,
    grid=(TGRID,),
    in_specs=[
        pl.BlockSpec((BR, F2H), lambda i: (i, 0)),
        pl.BlockSpec((BR, F2H), lambda i: (i, 0)),
        pl.BlockSpec((BR, F2), lambda i: (i, 0)),
        pl.BlockSpec((BR, 1), lambda i: (i, 0)),
        _full((1, 64)),
        _full((1, 128)),
        _full((1, 64)),
        _full((1, 64)),
        _full((1, 128)),
        _full((1, 128)),
        _full((8, 64, 64)),
        _full((8, 128, 64)),
        _full((64, 64)),
        _full((128, 64)),
        _full((1, 64)),
        _full((1, 64)),
        pl.BlockSpec(memory_space=pltpu.SMEM),
    ],
    out_specs=pl.BlockSpec((BR, OUT_CH), lambda i: (i, 0)),
    out_shape=jax.ShapeDtypeStruct((N_PAD, OUT_CH), jnp.float32),
)


def kernel(x, edge_index, edge_label_index, att_weights,
           p0_W1, p0_b1, p0_W2, p0_b2, p0_ln_g, p0_ln_b, p0_spline_W,
           p0_base_W, p0_base_b,
           p1_W1, p1_b1, p1_W2, p1_b2, p1_ln_g, p1_ln_b, p1_spline_W,
           p1_base_W, p1_base_b):
    f32 = jnp.float32
    x_pad = jnp.concatenate([x, jnp.zeros((N_PAD - N, IN_CH), f32)], axis=0)
    row3 = edge_index[0].reshape(NW, K, CH)
    col3 = edge_index[1].reshape(NW, K, CH)
    row3b = edge_index[0].reshape(16, K2, CH)
    col3b = edge_index[1].reshape(16, K2, CH)
    zCH = jnp.zeros((CH,), f32)
    oCH = jnp.ones((CH,), f32)
    z128 = jnp.zeros((CH, IN_CH), f32)
    z96 = jnp.zeros((CH, F2H), f32)

    degp = _deg(col3, zCH, oCH)
    d0 = degp[:N_PAD].reshape(N_PAD, 1)
    d1 = degp[N_PAD:].reshape(N_PAD, 1)
    dinv, g0 = _tc_pre(x_pad, d0, d1)

    s0 = _prop128e(g0, row3, col3, z128)
    g2 = _tc_mid(s0[:N_PAD], s0[N_PAD:], g0, dinv,
                 p0_W1, p0_b1.reshape(1, 64), p0_W2,
                 p1_W1, p1_b1.reshape(1, 128), p1_W2)

    g2v = g2.reshape(2 * N_PAD, F2H)
    s2 = _prop96(g2v, row3b, col3b, z96)
    sG0 = p0_spline_W.reshape(64, 8, 64).transpose(1, 0, 2)
    sG1 = p1_spline_W.reshape(128, 8, 64).transpose(1, 0, 2)
    z = _tc_post(s2[:N_PAD], s2[N_PAD:], g2, dinv,
                 p0_b2.reshape(1, 64), p1_b2.reshape(1, 128),
                 p0_ln_g.reshape(1, 64), p0_ln_b.reshape(1, 64),
                 p1_ln_g.reshape(1, 128), p1_ln_b.reshape(1, 128),
                 sG0, sG1, p0_base_W, p1_base_W,
                 p0_base_b.reshape(1, 64), p1_base_b.reshape(1, 64),
                 att_weights)

    pad = NL_PAD - NL
    padidx = (jnp.arange(pad, dtype=jnp.int32) * 13) % N
    src3 = jnp.concatenate([edge_label_index[0], padidx]).reshape(NW, KD, CD)
    dst3 = jnp.concatenate([edge_label_index[1], padidx]).reshape(NW, KD, CD)
    res = _decode(z, src3, dst3)
    return res[:NL]


# speedup vs baseline: 1.6023x; 1.0098x over previous
import functools

import jax
import jax.numpy as jnp
from jax import lax
from jax.experimental import pallas as pl
from jax.experimental.pallas import tpu as pltpu
from jax.experimental.pallas import tpu_sc as plsc

N = 10000
N_PAD = 10240
E = 320000
NL = 200000
IN_CH = 128
F2 = 192
OUT_CH = 64

NW = 32
EPW = E // NW
CH = 80
K = EPW // CH
EPT = E // 16
K2 = EPT // CH
RPS = N_PAD // 16
F2H = F2 // 2

CD = 64
WPD = 6272
KD = WPD // CD
NL_PAD = NW * WPD

BR = 512
TGRID = N_PAD // BR

_MESH = plsc.VectorSubcoreMesh(core_axis_name="c", subcore_axis_name="s")

GRID_PTS = [-2.0 + j * (4.0 / 7.0) for j in range(8)]
INV_DENOM = 7.0 / 4.0


@functools.partial(
    pl.kernel,
    out_type=jax.ShapeDtypeStruct((2 * N_PAD,), jnp.float32),
    mesh=_MESH,
    scratch_types=[
        pltpu.VMEM_SHARED((N_PAD,), jnp.float32),
        pltpu.VMEM((K, CH), jnp.int32),
        pltpu.VMEM((CH,), jnp.float32),
        pltpu.VMEM((CH,), jnp.float32),
        pltpu.SemaphoreType.DMA,
    ],
)
def _deg(col_hbm, zer_hbm, one_hbm, out_hbm, acc, coli, zbuf, obuf, sem):
    c = lax.axis_index("c")
    s = lax.axis_index("s")
    wid = c * 16 + s
    pltpu.sync_copy(zer_hbm, zbuf)
    pltpu.sync_copy(one_hbm, obuf)
    pltpu.sync_copy(col_hbm.at[wid], coli)
    rbase = s * RPS
    for j in range(RPS // CH):
        pltpu.sync_copy(zbuf, acc.at[pl.ds(rbase + j * CH, CH)])
    plsc.subcore_barrier()

    def body(k, carry):
        pltpu.sync_copy(obuf, acc.at[coli.at[k]], add=True)
        return carry

    lax.fori_loop(0, K, body, 0)
    plsc.subcore_barrier()
    for j in range(RPS // CH):
        r0 = rbase + j * CH
        pltpu.sync_copy(acc.at[pl.ds(r0, CH)], zbuf)
        pltpu.sync_copy(zbuf, out_hbm.at[pl.ds(c * N_PAD + r0, CH)])


def _make_prop(FH, NB, KC, feature_split, tc_tiling):
    @functools.partial(
        pl.kernel,
        out_type=jax.ShapeDtypeStruct((2 * N_PAD, FH), jnp.float32),
        mesh=_MESH,
        compiler_params=pltpu.CompilerParams(use_tc_tiling_on_sc=tc_tiling),
        scratch_types=[
            pltpu.VMEM_SHARED((N_PAD, FH), jnp.float32),
            pltpu.VMEM((KC, CH), jnp.int32),
            pltpu.VMEM((KC, CH), jnp.int32),
            [pltpu.VMEM((CH, FH), jnp.float32)] * NB,
            [pltpu.SemaphoreType.DMA] * NB,
            [pltpu.SemaphoreType.DMA] * NB,
        ],
    )
    def prop(g_hbm, row_hbm, col_hbm, zer_hbm, out_hbm, acc, rowi, coli,
             bufs, gs, ss):
        c = lax.axis_index("c")
        s = lax.axis_index("s")
        if feature_split:
            pltpu.sync_copy(row_hbm.at[s], rowi)
            pltpu.sync_copy(col_hbm.at[s], coli)

            def tbody(r, carry):
                for jj in range(CH // 16):
                    v = rowi[r, pl.ds(jj * 16, 16)]
                    rowi[r, pl.ds(jj * 16, 16)] = v * 2 + c
                return carry

            lax.fori_loop(0, KC, tbody, 0)
        else:
            wid = c * 16 + s
            pltpu.sync_copy(row_hbm.at[wid], rowi)
            pltpu.sync_copy(col_hbm.at[wid], coli)
        pltpu.sync_copy(zer_hbm, bufs[0])
        rbase = s * RPS
        NR = RPS // CH

        def accsl(j):
            return acc.at[pl.ds(rbase + j * CH, CH)]

        def outsl(j):
            return out_hbm.at[pl.ds(c * N_PAD + rbase + j * CH, CH)]

        for j in range(NR):
            pltpu.async_copy(bufs[0], accsl(j), gs[0])
        for j in range(NR):
            pltpu.make_async_copy(bufs[0], accsl(j), gs[0]).wait()
        plsc.subcore_barrier()

        def gstart(k, u):
            pltpu.async_copy(g_hbm.at[rowi.at[k]], bufs[u], gs[u])

        def gwait(k, u):
            pltpu.make_async_copy(g_hbm.at[rowi.at[k]], bufs[u], gs[u]).wait()

        def sstart(k, u):
            pltpu.async_copy(bufs[u], acc.at[coli.at[k]], ss[u], add=True)

        def swait(k, u):
            pltpu.make_async_copy(bufs[u], acc.at[coli.at[k]], ss[u]).wait()

        for u in range(NB):
            gstart(u, u)

        def body(i, carry):
            k0 = NB * i
            for u in range(NB):
                gwait(k0 + u, u)
                sstart(k0 + u, u)
            for u in range(NB):
                swait(k0 + u, u)
                gstart(k0 + u + NB, u)
            return carry

        lax.fori_loop(0, KC // NB - 1, body, 0)
        t0 = NB * (KC // NB - 1)
        for u in range(NB):
            gwait(t0 + u, u)
            sstart(t0 + u, u)
        for k in range(t0 + NB, KC):
            u = k % NB
            swait(k - NB, u)
            gstart(k, u)
            gwait(k, u)
            sstart(k, u)
        for k in range(max(t0, KC - NB), KC):
            swait(k, k % NB)
        plsc.subcore_barrier()
        pltpu.async_copy(accsl(0), bufs[0], gs[0])
        for j in range(NR):
            u = j % 2
            pltpu.make_async_copy(accsl(j), bufs[u], gs[u]).wait()
            pltpu.async_copy(bufs[u], outsl(j), ss[u])
            if j + 1 < NR:
                if j >= 1:
                    pltpu.make_async_copy(
                        bufs[1 - u], outsl(j - 1), ss[1 - u]).wait()
                pltpu.async_copy(accsl(j + 1), bufs[1 - u], gs[1 - u])
        for j in range(max(0, NR - 2), NR):
            pltpu.make_async_copy(bufs[j % 2], outsl(j), ss[j % 2]).wait()

    return prop


_prop128e = _make_prop(IN_CH, 2, K, False, False)
_prop96 = _make_prop(F2H, 3, K2, True, False)


@functools.partial(
    pl.kernel,
    out_type=jax.ShapeDtypeStruct((NL_PAD,), jnp.float32),
    mesh=_MESH,
    compiler_params=pltpu.CompilerParams(
        use_tc_tiling_on_sc=False, needs_layout_passes=False),
    scratch_types=[
        pltpu.VMEM((KD, CD), jnp.int32),
        pltpu.VMEM((KD, CD), jnp.int32),
        pltpu.VMEM((CD, OUT_CH), jnp.float32),
        pltpu.VMEM((CD, OUT_CH), jnp.float32),
        pltpu.VMEM((CD, OUT_CH), jnp.float32),
        pltpu.VMEM((CD, OUT_CH), jnp.float32),
        pltpu.VMEM((CD, OUT_CH), jnp.float32),
        pltpu.VMEM((CD, OUT_CH), jnp.float32),
        pltpu.VMEM((WPD,), jnp.float32),
        pltpu.SemaphoreType.DMA,
        pltpu.SemaphoreType.DMA,
        pltpu.SemaphoreType.DMA,
        pltpu.SemaphoreType.DMA,
        pltpu.SemaphoreType.DMA,
        pltpu.SemaphoreType.DMA,
    ],
)
def _decode(z_hbm, src_hbm, dst_hbm, out_hbm, srci, dsti, sbuf0, dbuf0,
            sbuf1, dbuf1, sbuf2, dbuf2, obuf, sA0, sB0, sA1, sB1, sA2, sB2):
    c = lax.axis_index("c")
    s = lax.axis_index("s")
    wid = c * 16 + s
    pltpu.sync_copy(src_hbm.at[wid], srci)
    pltpu.sync_copy(dst_hbm.at[wid], dsti)

    sbufs = (sbuf0, sbuf1, sbuf2)
    dbufs = (dbuf0, dbuf1, dbuf2)
    sA = (sA0, sA1, sA2)
    sB = (sB0, sB1, sB2)

    def gstart(k, u):
        pltpu.async_copy(z_hbm.at[srci.at[k]], sbufs[u], sA[u])
        pltpu.async_copy(z_hbm.at[dsti.at[k]], dbufs[u], sB[u])

    def gwait(k, u):
        pltpu.make_async_copy(z_hbm.at[srci.at[k]], sbufs[u], sA[u]).wait()
        pltpu.make_async_copy(z_hbm.at[dsti.at[k]], dbufs[u], sB[u]).wait()

    def compute(k, u):
        sb = sbufs[u]
        db = dbufs[u]

        def gb(g16, carry2):
            acc = jnp.zeros((16,), jnp.float32)
            for w in range(16):
                e = g16 * 16 + w
                p = None
                for j in range(OUT_CH // 16):
                    sv = sb[e, pl.ds(j * 16, 16)]
                    dv = db[e, pl.ds(j * 16, 16)]
                    p = sv * dv if p is None else p + sv * dv
                r = jnp.sum(p)
                acc = jnp.where(lax.iota(jnp.int32, 16) == w, r, acc)
            obuf[pl.ds(k * CD + g16 * 16, 16)] = acc
            return carry2

        lax.fori_loop(0, CD // 16, gb, 0)

    for u in range(3):
        gstart(u, u)

    def body(i, carry):
        k0 = 3 * i
        for u in range(3):
            gwait(k0 + u, u)
            compute(k0 + u, u)
            gstart(k0 + u + 3, u)
        return carry

    lax.fori_loop(0, KD // 3 - 1, body, 0)
    t0 = 3 * (KD // 3 - 1)
    for u in range(3):
        gwait(t0 + u, u)
        compute(t0 + u, u)
        if t0 + u + 3 < KD:
            gstart(t0 + u + 3, u)
    for k in range(t0 + 3, KD):
        gwait(k, k % 3)
        compute(k, k % 3)
    pltpu.sync_copy(obuf, out_hbm.at[pl.ds(wid * WPD, WPD)])


def _tc_pre_body(x_ref, d0_ref, d1_ref, dinv_ref, g0_ref):
    deg = d0_ref[...] + d1_ref[...] + 1.0
    dinv = lax.rsqrt(deg)
    dinv_ref[...] = dinv
    g0_ref[...] = dinv * x_ref[...]


_tc_pre = pl.pallas_call(
    _tc_pre_body,
    grid=(TGRID,),
    in_specs=[
        pl.BlockSpec((BR, IN_CH), lambda i: (i, 0)),
        pl.BlockSpec((BR, 1), lambda i: (i, 0)),
        pl.BlockSpec((BR, 1), lambda i: (i, 0)),
    ],
    out_specs=[
        pl.BlockSpec((BR, 1), lambda i: (i, 0)),
        pl.BlockSpec((BR, IN_CH), lambda i: (i, 0)),
    ],
    out_shape=[
        jax.ShapeDtypeStruct((N_PAD, 1), jnp.float32),
        jax.ShapeDtypeStruct((N_PAD, IN_CH), jnp.float32),
    ],
)


def _tc_mid_body(s0a, s0b, g0, dinv, W10, b10, W20, W11, b11, W21, g2_ref):
    dv = dinv[...]
    P = dv * (s0a[...] + s0b[...] + g0[...])
    h0 = jnp.dot(P, W10[...], preferred_element_type=jnp.float32) + b10[...]
    t0 = jnp.dot(h0, W20[...], preferred_element_type=jnp.float32)
    h1 = jnp.dot(P, W11[...], preferred_element_type=jnp.float32) + b11[...]
    t1 = jnp.dot(h1, W21[...], preferred_element_type=jnp.float32)
    g2_ref[...] = jnp.concatenate([dv * t0, dv * t1], axis=1)


def _full(shape):
    return pl.BlockSpec(shape, lambda i: tuple(0 for _ in shape))


_tc_mid = pl.pallas_call(
    _tc_mid_body,
    grid=(TGRID,),
    in_specs=[
        pl.BlockSpec((BR, IN_CH), lambda i: (i, 0)),
        pl.BlockSpec((BR, IN_CH), lambda i: (i, 0)),
        pl.BlockSpec((BR, IN_CH), lambda i: (i, 0)),
        pl.BlockSpec((BR, 1), lambda i: (i, 0)),
        _full((IN_CH, 64)),
        _full((1, 64)),
        _full((64, 64)),
        _full((IN_CH, 128)),
        _full((1, 128)),
        _full((128, 128)),
    ],
    out_specs=pl.BlockSpec((BR, F2), lambda i: (i, 0)),
    out_shape=jax.ShapeDtypeStruct((N_PAD, F2), jnp.float32),
)


def _fastkan(carr, ln_g, ln_b, sG, bW, bb):
    mu = jnp.mean(carr, axis=1, keepdims=True)
    xc = carr - mu
    var = jnp.mean(xc * xc, axis=1, keepdims=True)
    xn = xc * lax.rsqrt(var + 1e-5) * ln_g + ln_b
    spline = None
    for g in range(8):
        bas = jnp.exp(-((xn - GRID_PTS[g]) * INV_DENOM) ** 2)
        term = jnp.dot(bas, sG[g], preferred_element_type=jnp.float32)
        spline = term if spline is None else spline + term
    sig = 1.0 / (1.0 + jnp.exp(-carr))
    base = jnp.dot(carr * sig, bW, preferred_element_type=jnp.float32) + bb
    return spline + base


def _tc_post_body(s2a, s2b, g2, dinv, b20, b21,
                  lng0, lnb0, lng1, lnb1, sG0, sG1, bW0, bW1, bb0, bb1, att,
                  z_ref):
    dv = dinv[...]
    s2 = jnp.concatenate([s2a[...], s2b[...]], axis=1)
    c0 = dv * (s2[:, :64] + g2[:, :64]) + b20[...]
    c1 = dv * (s2[:, 64:] + g2[:, 64:]) + b21[...]
    out0 = _fastkan(c0, lng0[...], lnb0[...], sG0, bW0[...], bb0[...])
    out1 = _fastkan(c1, lng1[...], lnb1[...], sG1, bW1[...], bb1[...])
    a0 = att[0]
    a1 = att[1]
    m = jnp.maximum(a0, a1)
    e0 = jnp.exp(jnp.full((BR, OUT_CH), a0 - m, jnp.float32))
    e1 = jnp.exp(jnp.full((BR, OUT_CH), a1 - m, jnp.float32))
    z_ref[...] = (e0 * out0 + e1 * out1) / (e0 + e1)


_tc_post = pl.pallas_call(
    _tc_post_body,
    grid=(TGRID,),
    in_specs=[
        pl.BlockSpec((BR, F2H), lambda i: (i, 0)),
        pl.BlockSpec((BR, F2H), lambda i: (i, 0)),
        pl.BlockSpec((BR, F2), lambda i: (i, 0)),
        pl.BlockSpec((BR, 1), lambda i: (i, 0)),
        _full((1, 64)),
        _full((1, 128)),
        _full((1, 64)),
        _full((1, 64)),
        _full((1, 128)),
        _full((1, 128)),
        _full((8, 64, 64)),
        _full((8, 128, 64)),
        _full((64, 64)),
        _full((128, 64)),
        _full((1, 64)),
        _full((1, 64)),
        pl.BlockSpec(memory_space=pltpu.SMEM),
    ],
    out_specs=pl.BlockSpec((BR, OUT_CH), lambda i: (i, 0)),
    out_shape=jax.ShapeDtypeStruct((N_PAD, OUT_CH), jnp.float32),
)


def kernel(x, edge_index, edge_label_index, att_weights,
           p0_W1, p0_b1, p0_W2, p0_b2, p0_ln_g, p0_ln_b, p0_spline_W,
           p0_base_W, p0_base_b,
           p1_W1, p1_b1, p1_W2, p1_b2, p1_ln_g, p1_ln_b, p1_spline_W,
           p1_base_W, p1_base_b):
    f32 = jnp.float32
    x_pad = jnp.concatenate([x, jnp.zeros((N_PAD - N, IN_CH), f32)], axis=0)
    row3 = edge_index[0].reshape(NW, K, CH)
    col3 = edge_index[1].reshape(NW, K, CH)
    row3b = edge_index[0].reshape(16, K2, CH)
    col3b = edge_index[1].reshape(16, K2, CH)
    zCH = jnp.zeros((CH,), f32)
    oCH = jnp.ones((CH,), f32)
    z128 = jnp.zeros((CH, IN_CH), f32)
    z96 = jnp.zeros((CH, F2H), f32)

    degp = _deg(col3, zCH, oCH)
    d0 = degp[:N_PAD].reshape(N_PAD, 1)
    d1 = degp[N_PAD:].reshape(N_PAD, 1)
    dinv, g0 = _tc_pre(x_pad, d0, d1)

    s0 = _prop128e(g0, row3, col3, z128)
    g2 = _tc_mid(s0[:N_PAD], s0[N_PAD:], g0, dinv,
                 p0_W1, p0_b1.reshape(1, 64), p0_W2,
                 p1_W1, p1_b1.reshape(1, 128), p1_W2)

    g2v = g2.reshape(2 * N_PAD, F2H)
    s2 = _prop96(g2v, row3b, col3b, z96)
    sG0 = p0_spline_W.reshape(64, 8, 64).transpose(1, 0, 2)
    sG1 = p1_spline_W.reshape(128, 8, 64).transpose(1, 0, 2)
    z = _tc_post(s2[:N_PAD], s2[N_PAD:], g2, dinv,
                 p0_b2.reshape(1, 64), p1_b2.reshape(1, 128),
                 p0_ln_g.reshape(1, 64), p0_ln_b.reshape(1, 64),
                 p1_ln_g.reshape(1, 128), p1_ln_b.reshape(1, 128),
                 sG0, sG1, p0_base_W, p1_base_W,
                 p0_base_b.reshape(1, 64), p1_base_b.reshape(1, 64),
                 att_weights)

    pad = NL_PAD - NL
    padidx = (jnp.arange(pad, dtype=jnp.int32) * 13) % N
    src3 = jnp.concatenate([edge_label_index[0], padidx]).reshape(NW, KD, CD)
    dst3 = jnp.concatenate([edge_label_index[1], padidx]).reshape(NW, KD, CD)
    res = _decode(z, src3, dst3)
    return res[:NL]
